# baseline (device time: 263192 ns/iter reference)
import jax
import jax.numpy as jnp
from jax import lax
from jax.experimental import pallas as pl
from jax.experimental.pallas import tpu as pltpu

F_CHUNK = 256
N_CC = 16


def kernel(x, dy):
    m, d = x.shape
    _, f = dy.shape
    dq = d // 4
    n_mm = f // F_CHUNK
    cw = f // N_CC
    mm_per_cc = cw // F_CHUNK

    def body(x_ref, dy_ref, out_ref, xs, dyc, p_ref, recv1, recv2,
             xs_sem, load_sems, st1_sems, st2_sems,
             sa_s, sa_r, sb1_s, sb1_r, sb2_s, sb2_r, sb3_s, sb3_r):
        my_x = lax.axis_index("x")
        my_y = lax.axis_index("y")
        my_z = lax.axis_index("z")
        is_owner = my_x == my_y
        r0 = (2 * my_y + my_z) * dq

        x_peer = (1 - my_x, my_y, my_z)
        z_peer = (my_x, my_y, 1 - my_z)
        y_peer = (my_x, 1 - my_y, my_z)

        def cslice(ref, cc):
            return ref.at[:, pl.ds(cc * cw, cw)]

        def a_rdma(cc):
            return pltpu.make_async_remote_copy(
                src_ref=cslice(p_ref, cc), dst_ref=cslice(recv1, cc),
                send_sem=sa_s.at[cc], recv_sem=sa_r.at[cc],
                device_id=x_peer, device_id_type=pl.DeviceIdType.MESH)

        def b1_rdma(cc):
            return pltpu.make_async_remote_copy(
                src_ref=cslice(p_ref, cc), dst_ref=cslice(recv2, cc),
                send_sem=sb1_s.at[cc], recv_sem=sb1_r.at[cc],
                device_id=z_peer, device_id_type=pl.DeviceIdType.MESH)

        def b2_rdma(cc):
            return pltpu.make_async_remote_copy(
                src_ref=cslice(p_ref, cc), dst_ref=cslice(recv1, cc),
                send_sem=sb2_s.at[cc], recv_sem=sb2_r.at[cc],
                device_id=y_peer, device_id_type=pl.DeviceIdType.MESH)

        def b3_rdma(cc):
            return pltpu.make_async_remote_copy(
                src_ref=cslice(recv1, cc), dst_ref=cslice(recv2, cc),
                send_sem=sb3_s.at[cc], recv_sem=sb3_r.at[cc],
                device_id=z_peer, device_id_type=pl.DeviceIdType.MESH)

        def store1(src, cc):
            st = pltpu.make_async_copy(
                cslice(src, cc),
                out_ref.at[pl.ds(my_z * dq, dq), pl.ds(cc * cw, cw)],
                st1_sems.at[cc])
            st.start()
            return st

        def store2(cc):
            st = pltpu.make_async_copy(
                cslice(recv2, cc),
                out_ref.at[pl.ds((1 - my_z) * dq, dq), pl.ds(cc * cw, cw)],
                st2_sems.at[cc])
            st.start()
            return st

        def owner_reduce_and_forward(cc):
            a_rdma(cc).wait_recv()
            sl = pl.ds(cc * cw, cw)
            p_ref[:, sl] = p_ref[:, sl] + recv1[:, sl]
            b1_rdma(cc).start()
            b2_rdma(cc).start()
            store1(p_ref, cc)

        xcp = pltpu.make_async_copy(x_ref.at[:, pl.ds(r0, dq)], xs, xs_sem)
        xcp.start()

        def dy_load(c):
            return pltpu.make_async_copy(
                dy_ref.at[:, pl.ds(c * F_CHUNK, F_CHUNK)],
                dyc.at[c % 2], load_sems.at[c % 2])

        dy_load(0).start()
        xcp.wait()

        for c in range(n_mm):
            if c + 1 < n_mm:
                dy_load(c + 1).start()
            dy_load(c).wait()
            p_ref[:, pl.ds(c * F_CHUNK, F_CHUNK)] = lax.dot_general(
                xs[:, :], dyc[c % 2, :, :], (((0,), (0,)), ((), ())),
                preferred_element_type=jnp.float32)
            if (c + 1) % mm_per_cc == 0:
                cc = (c + 1) // mm_per_cc - 1

                @pl.when(jnp.logical_not(is_owner))
                def _():
                    a_rdma(cc).start()

                if cc >= 1:
                    @pl.when(is_owner)
                    def _():
                        owner_reduce_and_forward(cc - 1)

        @pl.when(is_owner)
        def _():
            owner_reduce_and_forward(N_CC - 1)
            for cc in range(N_CC):
                b1_rdma(cc).wait_recv()
                store2(cc)
            for cc in range(N_CC):
                b1_rdma(cc).wait_send()
                b2_rdma(cc).wait_send()

        @pl.when(jnp.logical_not(is_owner))
        def _():
            for cc in range(N_CC):
                b2_rdma(cc).wait_recv()
                b3_rdma(cc).start()
                store1(recv1, cc)
            for cc in range(N_CC):
                b3_rdma(cc).wait_recv()
                store2(cc)
            for cc in range(N_CC):
                a_rdma(cc).wait_send()
                b3_rdma(cc).wait_send()

        for cc in range(N_CC):
            pltpu.make_async_copy(
                cslice(recv2, cc),
                out_ref.at[pl.ds(my_z * dq, dq), pl.ds(cc * cw, cw)],
                st1_sems.at[cc]).wait()
            pltpu.make_async_copy(
                cslice(recv2, cc),
                out_ref.at[pl.ds((1 - my_z) * dq, dq), pl.ds(cc * cw, cw)],
                st2_sems.at[cc]).wait()

    return pl.pallas_call(
        body,
        out_shape=jax.ShapeDtypeStruct((d // 2, f), jnp.float32),
        in_specs=[
            pl.BlockSpec(memory_space=pl.ANY),
            pl.BlockSpec(memory_space=pl.ANY),
        ],
        out_specs=pl.BlockSpec(memory_space=pl.ANY),
        scratch_shapes=[
            pltpu.VMEM((m, dq), jnp.float32),
            pltpu.VMEM((2, m, F_CHUNK), jnp.float32),
            pltpu.VMEM((dq, f), jnp.float32),
            pltpu.VMEM((dq, f), jnp.float32),
            pltpu.VMEM((dq, f), jnp.float32),
            pltpu.SemaphoreType.DMA,
            pltpu.SemaphoreType.DMA((2,)),
            pltpu.SemaphoreType.DMA((N_CC,)),
            pltpu.SemaphoreType.DMA((N_CC,)),
            pltpu.SemaphoreType.DMA((N_CC,)),
            pltpu.SemaphoreType.DMA((N_CC,)),
            pltpu.SemaphoreType.DMA((N_CC,)),
            pltpu.SemaphoreType.DMA((N_CC,)),
            pltpu.SemaphoreType.DMA((N_CC,)),
            pltpu.SemaphoreType.DMA((N_CC,)),
            pltpu.SemaphoreType.DMA((N_CC,)),
            pltpu.SemaphoreType.DMA((N_CC,)),
        ],
        compiler_params=pltpu.CompilerParams(
            vmem_limit_bytes=60 * 1024 * 1024,
        ),
    )(x, dy)


# device time: 262634 ns/iter; 1.0021x vs baseline; 1.0021x over previous
import jax
import jax.numpy as jnp
from jax import lax
from jax.experimental import pallas as pl
from jax.experimental.pallas import tpu as pltpu

F_CHUNK = 256
N_CC = 16


def kernel(x, dy):
    m, d = x.shape
    _, f = dy.shape
    dq = d // 4
    n_mm = f // F_CHUNK
    cw = f // N_CC
    mm_per_cc = cw // F_CHUNK

    def body(x_ref, dy_ref, out_ref, xs, dyc, p_ref, recv1, recv2,
             xs_sem, load_sems, st1_sems, st2_sems,
             sa_s, sa_r, sb1_s, sb1_r, sb2_s, sb2_r, sb3_s, sb3_r):
        my_x = lax.axis_index("x")
        my_y = lax.axis_index("y")
        my_z = lax.axis_index("z")
        is_owner = my_x == my_y
        r0 = (2 * my_y + my_z) * dq

        x_peer = (1 - my_x, my_y, my_z)
        z_peer = (my_x, my_y, 1 - my_z)
        y_peer = (my_x, 1 - my_y, my_z)

        def a_rdma(cc):
            return pltpu.make_async_remote_copy(
                src_ref=p_ref.at[cc], dst_ref=recv1.at[cc],
                send_sem=sa_s.at[cc], recv_sem=sa_r.at[cc],
                device_id=x_peer, device_id_type=pl.DeviceIdType.MESH)

        def b1_rdma(cc):
            return pltpu.make_async_remote_copy(
                src_ref=p_ref.at[cc], dst_ref=recv2.at[cc],
                send_sem=sb1_s.at[cc], recv_sem=sb1_r.at[cc],
                device_id=z_peer, device_id_type=pl.DeviceIdType.MESH)

        def b2_rdma(cc):
            return pltpu.make_async_remote_copy(
                src_ref=p_ref.at[cc], dst_ref=recv1.at[cc],
                send_sem=sb2_s.at[cc], recv_sem=sb2_r.at[cc],
                device_id=y_peer, device_id_type=pl.DeviceIdType.MESH)

        def b3_rdma(cc):
            return pltpu.make_async_remote_copy(
                src_ref=recv1.at[cc], dst_ref=recv2.at[cc],
                send_sem=sb3_s.at[cc], recv_sem=sb3_r.at[cc],
                device_id=z_peer, device_id_type=pl.DeviceIdType.MESH)

        def store1(src, cc):
            pltpu.make_async_copy(
                src.at[cc],
                out_ref.at[pl.ds(my_z * dq, dq), pl.ds(cc * cw, cw)],
                st1_sems.at[cc]).start()

        def store2(cc):
            pltpu.make_async_copy(
                recv2.at[cc],
                out_ref.at[pl.ds((1 - my_z) * dq, dq), pl.ds(cc * cw, cw)],
                st2_sems.at[cc]).start()

        def owner_reduce_and_forward(cc):
            a_rdma(cc).wait_recv()
            p_ref[cc] = p_ref[cc] + recv1[cc]
            b1_rdma(cc).start()
            b2_rdma(cc).start()
            store1(p_ref, cc)

        xcp = pltpu.make_async_copy(x_ref.at[:, pl.ds(r0, dq)], xs, xs_sem)
        xcp.start()

        def dy_load(c):
            return pltpu.make_async_copy(
                dy_ref.at[:, pl.ds(c * F_CHUNK, F_CHUNK)],
                dyc.at[c % 2], load_sems.at[c % 2])

        dy_load(0).start()
        xcp.wait()

        for c in range(n_mm):
            if c + 1 < n_mm:
                dy_load(c + 1).start()
            dy_load(c).wait()
            pc = lax.dot_general(
                xs[:, :], dyc[c % 2, :, :], (((0,), (0,)), ((), ())),
                preferred_element_type=jnp.float32)
            ci = c // mm_per_cc
            if mm_per_cc == 1:
                p_ref[ci] = pc
            else:
                p_ref[ci, :, pl.ds((c % mm_per_cc) * F_CHUNK, F_CHUNK)] = pc
            if (c + 1) % mm_per_cc == 0:
                cc = ci

                @pl.when(jnp.logical_not(is_owner))
                def _():
                    a_rdma(cc).start()

                if cc >= 1:
                    @pl.when(is_owner)
                    def _():
                        owner_reduce_and_forward(cc - 1)

        @pl.when(is_owner)
        def _():
            owner_reduce_and_forward(N_CC - 1)
            for cc in range(N_CC):
                b1_rdma(cc).wait_recv()
                store2(cc)
            for cc in range(N_CC):
                b1_rdma(cc).wait_send()
                b2_rdma(cc).wait_send()

        @pl.when(jnp.logical_not(is_owner))
        def _():
            for cc in range(N_CC):
                b2_rdma(cc).wait_recv()
                b3_rdma(cc).start()
                store1(recv1, cc)
            for cc in range(N_CC):
                b3_rdma(cc).wait_recv()
                store2(cc)
            for cc in range(N_CC):
                a_rdma(cc).wait_send()
                b3_rdma(cc).wait_send()

        for cc in range(N_CC):
            pltpu.make_async_copy(
                recv2.at[cc],
                out_ref.at[pl.ds(my_z * dq, dq), pl.ds(cc * cw, cw)],
                st1_sems.at[cc]).wait()
            pltpu.make_async_copy(
                recv2.at[cc],
                out_ref.at[pl.ds((1 - my_z) * dq, dq), pl.ds(cc * cw, cw)],
                st2_sems.at[cc]).wait()

    return pl.pallas_call(
        body,
        out_shape=jax.ShapeDtypeStruct((d // 2, f), jnp.float32),
        in_specs=[
            pl.BlockSpec(memory_space=pl.ANY),
            pl.BlockSpec(memory_space=pl.ANY),
        ],
        out_specs=pl.BlockSpec(memory_space=pl.ANY),
        scratch_shapes=[
            pltpu.VMEM((m, dq), jnp.float32),
            pltpu.VMEM((2, m, F_CHUNK), jnp.float32),
            pltpu.VMEM((N_CC, dq, cw), jnp.float32),
            pltpu.VMEM((N_CC, dq, cw), jnp.float32),
            pltpu.VMEM((N_CC, dq, cw), jnp.float32),
            pltpu.SemaphoreType.DMA,
            pltpu.SemaphoreType.DMA((2,)),
            pltpu.SemaphoreType.DMA((N_CC,)),
            pltpu.SemaphoreType.DMA((N_CC,)),
            pltpu.SemaphoreType.DMA((N_CC,)),
            pltpu.SemaphoreType.DMA((N_CC,)),
            pltpu.SemaphoreType.DMA((N_CC,)),
            pltpu.SemaphoreType.DMA((N_CC,)),
            pltpu.SemaphoreType.DMA((N_CC,)),
            pltpu.SemaphoreType.DMA((N_CC,)),
            pltpu.SemaphoreType.DMA((N_CC,)),
            pltpu.SemaphoreType.DMA((N_CC,)),
        ],
        compiler_params=pltpu.CompilerParams(
            vmem_limit_bytes=60 * 1024 * 1024,
        ),
    )(x, dy)
